# Initial kernel scaffold; baseline (speedup 1.0000x reference)
import jax
import jax.numpy as jnp
from jax.experimental import pallas as pl


def kernel(adj_indices, adj_values, keepRate, dEmbeds, mEmbeds, disEmbeds, dHyper, mHyper, disHyper):
    raise NotImplementedError("write your pallas kernel here")


# speedup vs baseline: 3.0033x; 3.0033x over previous
import functools

import jax
import jax.numpy as jnp
from jax import lax
from jax.experimental import pallas as pl
from jax.experimental.pallas import tpu as pltpu
from jax.experimental.pallas import tpu_sc as plsc

N_DRUG = 4000
N_MIC = 4000
N_DIS = 2000
N = N_DRUG + N_MIC + N_DIS
D = 256
H = 128
HP = H // 2
E = 160000
NUM_TILES = 16
CH = 80
CPT = 128
GRP = 16
E_PAD = NUM_TILES * CPT * CH
RPT = 624
RPT_LAST = N - 15 * RPT

RB = 1000
NBLK = N // RB


def _build_spmm():
    mesh = plsc.VectorSubcoreMesh(core_axis_name="c", subcore_axis_name="s")

    @functools.partial(
        pl.kernel,
        out_type=[jax.ShapeDtypeStruct((N, H), jnp.float32),
                  jax.ShapeDtypeStruct((N, H), jnp.float32)],
        mesh=mesh,
        compiler_params=pltpu.CompilerParams(use_tc_tiling_on_sc=False,
                                             needs_layout_passes=False),
        scratch_types=[
            pltpu.VMEM((GRP, CH), jnp.int32),
            pltpu.VMEM((GRP, CH), jnp.int32),
            pltpu.VMEM((GRP, CH), jnp.float32),
            pltpu.VMEM((CH, HP), jnp.int32),
            pltpu.VMEM((CH, HP), jnp.int32),
            pltpu.VMEM((CH, H), jnp.float32),
            pltpu.VMEM_SHARED((N, H), jnp.float32),
            pltpu.SemaphoreType.DMA,
            pltpu.SemaphoreType.DMA,
            pltpu.SemaphoreType.DMA,
        ],
    )
    def spmm(src_hbm, dst_hbm, val_hbm, zeros_hbm, pk0_hbm, pk1_hbm,
             out0_hbm, out1_hbm, src_v, dst_v, val_v, pk_a, pk_b, scl, acc,
             sem_ga, sem_gb, sem_s):
        c = lax.axis_index("c")
        t = lax.axis_index("s")

        @pl.when(t < 15)
        def _():
            pltpu.sync_copy(zeros_hbm.at[pl.ds(0, RPT)],
                            acc.at[pl.ds(t * RPT, RPT)])

        @pl.when(t == 15)
        def _():
            pltpu.sync_copy(zeros_hbm, acc.at[pl.ds(15 * RPT, RPT_LAST)])

        plsc.subcore_barrier()

        def run(pk_hbm):
            def gather_start(j, buf, sem):
                pltpu.async_copy(pk_hbm.at[src_v.at[j]], buf, sem)

            def gather_wait(j, buf, sem):
                pltpu.make_async_copy(pk_hbm.at[src_v.at[j]], buf,
                                      sem).wait()

            def scatter(j):
                pltpu.async_copy(scl, acc.at[dst_v.at[j]], sem_s,
                                 add=True).wait()

            def scale(j, buf):
                def scale_body(g, carry2):
                    v16 = val_v[j, pl.ds(g * 16, 16)]
                    for lane in range(16):
                        bv = v16[lane]
                        r = g * 16 + lane
                        for k in range(4):
                            x32 = buf[r, pl.ds(k * 16, 16)]
                            xb = plsc.bitcast(x32, jnp.bfloat16)
                            a, b = plsc.unpack(
                                xb, format=plsc.PackFormat.INTERLEAVED)
                            scl[r, pl.ds(k * 32, 16)] = a * bv
                            scl[r, pl.ds(k * 32 + 16, 16)] = b * bv
                    return carry2

                lax.fori_loop(0, CH // 16, scale_body, 0)

            NP = GRP // 2

            def stage_body(gi, carry0):
                base = t * CPT + gi * GRP
                pltpu.sync_copy(src_hbm.at[pl.ds(base, GRP)], src_v)
                pltpu.sync_copy(dst_hbm.at[pl.ds(base, GRP)], dst_v)
                pltpu.sync_copy(val_hbm.at[pl.ds(base, GRP)], val_v)

                gather_start(0, pk_a, sem_ga)

                def pair_body(p, carry):
                    c0 = 2 * p
                    gather_start(c0 + 1, pk_b, sem_gb)
                    gather_wait(c0, pk_a, sem_ga)
                    scale(c0, pk_a)
                    scatter(c0)

                    @pl.when(p < NP - 1)
                    def _():
                        gather_start(c0 + 2, pk_a, sem_ga)

                    gather_wait(c0 + 1, pk_b, sem_gb)
                    scale(c0 + 1, pk_b)
                    scatter(c0 + 1)
                    return carry

                lax.fori_loop(0, NP, pair_body, 0)
                return carry0

            lax.fori_loop(0, CPT // GRP, stage_body, 0)

        @pl.when(c == 0)
        def _():
            run(pk0_hbm)

        @pl.when(c == 1)
        def _():
            run(pk1_hbm)

        plsc.subcore_barrier()

        def writeback(out_hbm):
            @pl.when(t < 15)
            def _():
                pltpu.sync_copy(acc.at[pl.ds(t * RPT, RPT)],
                                out_hbm.at[pl.ds(t * RPT, RPT)])

            @pl.when(t == 15)
            def _():
                pltpu.sync_copy(acc.at[pl.ds(15 * RPT, RPT_LAST)],
                                out_hbm.at[pl.ds(15 * RPT, RPT_LAST)])

        @pl.when(c == 0)
        def _():
            writeback(out0_hbm)

        @pl.when(c == 1)
        def _():
            writeback(out1_hbm)

    return spmm


_spmm_cache = []


def _get_spmm():
    if not _spmm_cache:
        _spmm_cache.append(_build_spmm())
    return _spmm_cache[0]


def _pack_half(x):
    b = x.astype(jnp.bfloat16).reshape(N, 4, 2, 16)
    u = lax.bitcast_convert_type(b, jnp.uint16)
    p = u[:, :, 0, :].astype(jnp.uint32) | (
        u[:, :, 1, :].astype(jnp.uint32) << 16)
    return lax.bitcast_convert_type(p, jnp.int32).reshape(N, HP)


def _type_of_block(i):
    return i // 4


def _tc_init(embeds, hypers):
    def body(emb_ref, hyp_ref, p_ref):
        p_ref[...] = jnp.dot(emb_ref[...], hyp_ref[0],
                             preferred_element_type=jnp.float32)

    return pl.pallas_call(
        body,
        grid=(NBLK,),
        in_specs=[pl.BlockSpec((RB, D), lambda i: (i, 0)),
                  pl.BlockSpec((1, D, H), lambda i: (_type_of_block(i), 0, 0))],
        out_specs=pl.BlockSpec((RB, H), lambda i: (i, 0)),
        out_shape=jax.ShapeDtypeStruct((N, H), jnp.float32),
    )(embeds, hypers)


def _tc_phase_a(p, prev):
    def body(p_ref, pv_ref, tmp_ref):
        i = pl.program_id(0)
        contrib = lax.dot_general(p_ref[...], pv_ref[...],
                                  (((0,), (0,)), ((), ())),
                                  preferred_element_type=jnp.float32)
        first = (i == 0) | (i == 4) | (i == 8)

        @pl.when(first)
        def _():
            tmp_ref[0] = contrib

        @pl.when(jnp.logical_not(first))
        def _():
            tmp_ref[0] += contrib

    return pl.pallas_call(
        body,
        grid=(NBLK,),
        in_specs=[pl.BlockSpec((RB, H), lambda i: (i, 0)),
                  pl.BlockSpec((RB, D), lambda i: (i, 0))],
        out_specs=pl.BlockSpec((1, H, D), lambda i: (_type_of_block(i), 0, 0)),
        out_shape=jax.ShapeDtypeStruct((3, H, D), jnp.float32),
    )(p, prev)


def _l2n(x):
    return x / (jnp.sqrt(jnp.sum(x * x, axis=1, keepdims=True)) + 1e-12)


def _tc_phase_b(p, tmp, sp0, sp1, extras=None):
    final = extras is not None

    def body(*refs):
        if final:
            (p_ref, tmp_ref, sp0_ref, sp1_ref, emb_ref, n1_ref,
             gcn_ref, hyp_ref, nx_ref, out_ref) = refs
        else:
            (p_ref, tmp_ref, sp0_ref, sp1_ref,
             gcn_ref, hyp_ref, nx_ref) = refs
        h = jnp.dot(p_ref[...], tmp_ref[0], preferred_element_type=jnp.float32)
        h = _l2n(h)
        g = _l2n(jnp.concatenate([sp0_ref[...], sp1_ref[...]], axis=1))
        gcn_ref[...] = g
        hyp_ref[...] = h
        nxt = g + h
        nx_ref[...] = nxt
        if final:
            out_ref[...] = emb_ref[...] + n1_ref[...] + nxt

    spec_rb_h = pl.BlockSpec((RB, H), lambda i: (i, 0))
    spec_rb_d = pl.BlockSpec((RB, D), lambda i: (i, 0))
    in_specs = [spec_rb_h,
                pl.BlockSpec((1, H, D), lambda i: (_type_of_block(i), 0, 0)),
                spec_rb_h, spec_rb_h]
    out_specs = [spec_rb_d, spec_rb_d, spec_rb_d]
    out_shape = [jax.ShapeDtypeStruct((N, D), jnp.float32),
                 jax.ShapeDtypeStruct((N, D), jnp.float32),
                 jax.ShapeDtypeStruct((N, D), jnp.float32)]
    args = [p, tmp, sp0, sp1]
    if final:
        in_specs += [spec_rb_d, spec_rb_d]
        out_specs = out_specs + [spec_rb_d]
        out_shape = out_shape + [jax.ShapeDtypeStruct((N, D), jnp.float32)]
        args += list(extras)

    return pl.pallas_call(
        body,
        grid=(NBLK,),
        in_specs=in_specs,
        out_specs=out_specs,
        out_shape=out_shape,
    )(*args)


def kernel(adj_indices, adj_values, keepRate, dEmbeds, mEmbeds, disEmbeds,
           dHyper, mHyper, disHyper):
    del keepRate
    pad = E_PAD - E
    src = jnp.pad(adj_indices[1].astype(jnp.int32), (0, pad)).reshape(-1, CH)
    dst = jnp.pad(adj_indices[0].astype(jnp.int32), (0, pad)).reshape(-1, CH)
    vals = jnp.pad(adj_values.astype(jnp.float32), (0, pad)).reshape(-1, CH)
    zeros = jnp.zeros((RPT_LAST, H), jnp.float32)

    embeds = jnp.concatenate([dEmbeds, mEmbeds, disEmbeds], axis=0)
    hypers = jnp.stack([dHyper, mHyper, disHyper])

    p = _tc_init(embeds, hypers)
    spmm_fn = _get_spmm()

    s0, s1 = spmm_fn(src, dst, vals, zeros,
                     _pack_half(embeds[:, :H]), _pack_half(embeds[:, H:]))
    tmp = _tc_phase_a(p, embeds)
    gcn1, hyper1, n1 = _tc_phase_b(p, tmp, s0, s1)

    s0, s1 = spmm_fn(src, dst, vals, zeros,
                     _pack_half(n1[:, :H]), _pack_half(n1[:, H:]))
    tmp = _tc_phase_a(p, n1)
    gcn2, hyper2, _, out = _tc_phase_b(p, tmp, s0, s1, extras=(embeds, n1))

    return (out, embeds, gcn1, gcn2, embeds, hyper1, hyper2)


# speedup vs baseline: 3.4627x; 1.1530x over previous
import functools

import jax
import jax.numpy as jnp
from jax import lax
from jax.experimental import pallas as pl
from jax.experimental.pallas import tpu as pltpu
from jax.experimental.pallas import tpu_sc as plsc

N_DRUG = 4000
N_MIC = 4000
N_DIS = 2000
N = N_DRUG + N_MIC + N_DIS
D = 256
H = 128
E = 160000
NUM_TILES = 16
HALF = N // 2

EPT = E // NUM_TILES
SCAN = 2000
CH = 64
GRP = 16
EG = GRP * CH
CAPC = 176
CAP = CAPC * CH

ART = 304
ART_LAST = HALF - 15 * ART

RB = 1000
NBLK = N // RB

_SC_PARAMS = pltpu.CompilerParams(use_tc_tiling_on_sc=False,
                                  needs_layout_passes=False)


def _build_partition():
    mesh = plsc.VectorSubcoreMesh(core_axis_name="c", subcore_axis_name="s")

    @functools.partial(
        pl.kernel,
        out_type=[jax.ShapeDtypeStruct((2, NUM_TILES, CAPC, CH), jnp.int32),
                  jax.ShapeDtypeStruct((2, NUM_TILES, CAPC, CH), jnp.int32),
                  jax.ShapeDtypeStruct((2, NUM_TILES, CAPC, CH), jnp.float32),
                  jax.ShapeDtypeStruct((2, NUM_TILES, 16), jnp.int32)],
        mesh=mesh,
        compiler_params=_SC_PARAMS,
        scratch_types=[
            pltpu.VMEM((SCAN,), jnp.int32),
            pltpu.VMEM((SCAN,), jnp.int32),
            pltpu.VMEM((SCAN,), jnp.float32),
            pltpu.VMEM((CAP,), jnp.int32),
            pltpu.VMEM((CAP,), jnp.int32),
            pltpu.VMEM((CAP,), jnp.float32),
            pltpu.VMEM((CAPC, CH), jnp.int32),
            pltpu.VMEM((CAPC, CH), jnp.int32),
            pltpu.VMEM((CAPC, CH), jnp.float32),
            pltpu.VMEM((16,), jnp.int32),
        ],
    )
    def part(src_hbm, dst_hbm, val_hbm, csrc_hbm, cdst_hbm, cval_hbm,
             cnt_hbm, s_v, d_v, v_v, cs1, cd1, cv1, cs2, cd2, cv2, cnt_v):
        c = lax.axis_index("c")
        t = lax.axis_index("s")
        lo = c * HALF
        lane = lax.iota(jnp.int32, 16)

        def dyn_gather(x, idx):
            return lax.gather(
                x, idx[:, None],
                lax.GatherDimensionNumbers(offset_dims=(),
                                           collapsed_slice_dims=(0,),
                                           start_index_map=(0,)),
                (1,), mode=lax.GatherScatterMode.PROMISE_IN_BOUNDS)

        def psum16(x):
            s = x
            for k in (1, 2, 4, 8):
                g = dyn_gather(s, jnp.maximum(lane - k, 0))
                s = s + jnp.where(lane >= k, g, 0)
            return s

        def blk_body(b, cnt):
            base = t * EPT + b * SCAN
            pltpu.sync_copy(src_hbm.at[pl.ds(base, SCAN)], s_v)
            pltpu.sync_copy(dst_hbm.at[pl.ds(base, SCAN)], d_v)
            pltpu.sync_copy(val_hbm.at[pl.ds(base, SCAN)], v_v)

            def vec_body(i, cnt2):
                sl = pl.ds(i * 16, 16)
                dl = d_v[sl] - lo
                m = (dl >= 0) & (dl < HALF)
                s = psum16(m.astype(jnp.int32))
                kk = lane + 1
                cl = jnp.zeros((16,), jnp.int32)
                for st in (8, 4, 2, 1):
                    sv = dyn_gather(s, cl + (st - 1))
                    cl = cl + jnp.where(sv < kk, st, 0)
                w = pl.ds(cnt2, 16)
                cs1[w] = dyn_gather(s_v[sl], cl)
                cd1[w] = dyn_gather(dl, cl)
                cv1[w] = dyn_gather(v_v[sl], cl)
                return cnt2 + s[15]

            return lax.fori_loop(0, SCAN // 16, vec_body, cnt)

        cnt = lax.fori_loop(0, EPT // SCAN, blk_body, jnp.int32(0))

        z_i = jnp.zeros((16,), jnp.int32)
        z_f = jnp.zeros((16,), jnp.float32)
        for k in range(EG // 16):
            w = pl.ds(cnt + k * 16, 16)
            cs1[w] = z_i
            cd1[w] = z_i
            cv1[w] = z_f

        def rep_body(r, carry):
            for k in range(CH // 16):
                sl = pl.ds(r * CH + k * 16, 16)
                s2 = pl.ds(k * 16, 16)
                cs2[r, s2] = cs1[sl]
                cd2[r, s2] = cd1[sl]
                cv2[r, s2] = cv1[sl]
            return carry

        lax.fori_loop(0, CAPC, rep_body, 0)

        cnt_v[...] = jnp.broadcast_to(cnt, (16,))
        pltpu.sync_copy(cs2, csrc_hbm.at[c, t])
        pltpu.sync_copy(cd2, cdst_hbm.at[c, t])
        pltpu.sync_copy(cv2, cval_hbm.at[c, t])
        pltpu.sync_copy(cnt_v, cnt_hbm.at[c, t])

    return part


def _build_spmm():
    mesh = plsc.VectorSubcoreMesh(core_axis_name="c", subcore_axis_name="s")

    @functools.partial(
        pl.kernel,
        out_type=[jax.ShapeDtypeStruct((N, H), jnp.float32),
                  jax.ShapeDtypeStruct((N, H), jnp.float32)],
        mesh=mesh,
        compiler_params=_SC_PARAMS,
        scratch_types=[
            pltpu.VMEM((GRP, CH), jnp.int32),
            pltpu.VMEM((GRP, CH), jnp.int32),
            pltpu.VMEM((GRP, CH), jnp.float32),
            pltpu.VMEM((CH, H), jnp.int32),
            pltpu.VMEM((CH, H), jnp.int32),
            pltpu.VMEM((CH, H), jnp.float32),
            pltpu.VMEM_SHARED((HALF, H), jnp.float32),
            pltpu.VMEM_SHARED((HALF, H), jnp.float32),
            pltpu.SemaphoreType.DMA,
            pltpu.SemaphoreType.DMA,
            pltpu.SemaphoreType.DMA,
        ],
    )
    def spmm(csrc_hbm, cdst_hbm, cval_hbm, cnt_hbm, zeros_hbm, pk_hbm,
             out0_hbm, out1_hbm, src_v, dst_v, val_v, pk_a, pk_b, scl,
             acc0, acc1, sem_ga, sem_gb, sem_s):
        c = lax.axis_index("c")
        t = lax.axis_index("s")

        @pl.when(t < 15)
        def _():
            pltpu.sync_copy(zeros_hbm.at[pl.ds(0, ART)],
                            acc0.at[pl.ds(t * ART, ART)])
            pltpu.sync_copy(zeros_hbm.at[pl.ds(0, ART)],
                            acc1.at[pl.ds(t * ART, ART)])

        @pl.when(t == 15)
        def _():
            pltpu.sync_copy(zeros_hbm, acc0.at[pl.ds(15 * ART, ART_LAST)])
            pltpu.sync_copy(zeros_hbm, acc1.at[pl.ds(15 * ART, ART_LAST)])

        pltpu.sync_copy(cnt_hbm.at[c, t], src_v.at[0, pl.ds(0, 16)])
        cnt = src_v[0, pl.ds(0, 16)][0]
        n_grp = (cnt + (EG - 1)) // EG
        plsc.subcore_barrier()

        def gather_start(j, buf, sem):
            pltpu.async_copy(pk_hbm.at[src_v.at[j]], buf, sem)

        def gather_wait(j, buf, sem):
            pltpu.make_async_copy(pk_hbm.at[src_v.at[j]], buf, sem).wait()

        def scatter(j, acc):
            pltpu.async_copy(scl, acc.at[dst_v.at[j]], sem_s,
                             add=True).wait()

        def scale_half(j, buf, half):
            def scale_body(g, carry2):
                v16 = val_v[j, pl.ds(g * 16, 16)]
                for lane in range(16):
                    bv = v16[lane]
                    r = g * 16 + lane
                    for k in range(4):
                        x32 = buf[r, pl.ds(half * 64 + k * 16, 16)]
                        xb = plsc.bitcast(x32, jnp.bfloat16)
                        a, b = plsc.unpack(
                            xb, format=plsc.PackFormat.INTERLEAVED)
                        scl[r, pl.ds(k * 32, 16)] = a * bv
                        scl[r, pl.ds(k * 32 + 16, 16)] = b * bv
                return carry2

            lax.fori_loop(0, CH // 16, scale_body, 0)

        def process(j, buf, sem):
            gather_wait(j, buf, sem)
            scale_half(j, buf, 0)
            scatter(j, acc0)
            scale_half(j, buf, 1)
            scatter(j, acc1)

        NP = GRP // 2

        def grp_body(gi, carry0):
            pltpu.sync_copy(csrc_hbm.at[c, t, pl.ds(gi * GRP, GRP)], src_v)
            pltpu.sync_copy(cdst_hbm.at[c, t, pl.ds(gi * GRP, GRP)], dst_v)
            pltpu.sync_copy(cval_hbm.at[c, t, pl.ds(gi * GRP, GRP)], val_v)

            gather_start(0, pk_a, sem_ga)

            def pair_body(p, carry):
                c0 = 2 * p
                gather_start(c0 + 1, pk_b, sem_gb)
                process(c0, pk_a, sem_ga)

                @pl.when(p < NP - 1)
                def _():
                    gather_start(c0 + 2, pk_a, sem_ga)

                process(c0 + 1, pk_b, sem_gb)
                return carry

            lax.fori_loop(0, NP, pair_body, 0)
            return carry0

        lax.fori_loop(0, n_grp, grp_body, 0)
        plsc.subcore_barrier()

        def writeback(acc, out_hbm):
            @pl.when(t < 15)
            def _():
                pltpu.sync_copy(
                    acc.at[pl.ds(t * ART, ART)],
                    out_hbm.at[pl.ds(c * HALF + t * ART, ART)])

            @pl.when(t == 15)
            def _():
                pltpu.sync_copy(
                    acc.at[pl.ds(15 * ART, ART_LAST)],
                    out_hbm.at[pl.ds(c * HALF + 15 * ART, ART_LAST)])

        writeback(acc0, out0_hbm)
        writeback(acc1, out1_hbm)

    return spmm


_sc_cache = {}


def _get_sc(name):
    if not _sc_cache:
        _sc_cache["part"] = _build_partition()
        _sc_cache["spmm"] = _build_spmm()
    return _sc_cache[name]


def _pack_rows(x):
    b = x.astype(jnp.bfloat16).reshape(N, 8, 2, 16)
    u = lax.bitcast_convert_type(b, jnp.uint16)
    p = u[:, :, 0, :].astype(jnp.uint32) | (
        u[:, :, 1, :].astype(jnp.uint32) << 16)
    return lax.bitcast_convert_type(p, jnp.int32).reshape(N, H)


def _type_of_block(i):
    return i // 4


def _tc_init(embeds, hypers):
    def body(emb_ref, hyp_ref, p_ref):
        p_ref[...] = jnp.dot(emb_ref[...], hyp_ref[0],
                             preferred_element_type=jnp.float32)

    return pl.pallas_call(
        body,
        grid=(NBLK,),
        in_specs=[pl.BlockSpec((RB, D), lambda i: (i, 0)),
                  pl.BlockSpec((1, D, H), lambda i: (_type_of_block(i), 0, 0))],
        out_specs=pl.BlockSpec((RB, H), lambda i: (i, 0)),
        out_shape=jax.ShapeDtypeStruct((N, H), jnp.float32),
    )(embeds, hypers)


def _tc_phase_a(p, prev):
    def body(p_ref, pv_ref, tmp_ref):
        i = pl.program_id(0)
        contrib = lax.dot_general(p_ref[...], pv_ref[...],
                                  (((0,), (0,)), ((), ())),
                                  preferred_element_type=jnp.float32)
        first = (i == 0) | (i == 4) | (i == 8)

        @pl.when(first)
        def _():
            tmp_ref[0] = contrib

        @pl.when(jnp.logical_not(first))
        def _():
            tmp_ref[0] += contrib

    return pl.pallas_call(
        body,
        grid=(NBLK,),
        in_specs=[pl.BlockSpec((RB, H), lambda i: (i, 0)),
                  pl.BlockSpec((RB, D), lambda i: (i, 0))],
        out_specs=pl.BlockSpec((1, H, D), lambda i: (_type_of_block(i), 0, 0)),
        out_shape=jax.ShapeDtypeStruct((3, H, D), jnp.float32),
    )(p, prev)


def _l2n(x):
    return x / (jnp.sqrt(jnp.sum(x * x, axis=1, keepdims=True)) + 1e-12)


def _tc_phase_b(p, tmp, sp0, sp1, extras=None):
    final = extras is not None

    def body(*refs):
        if final:
            (p_ref, tmp_ref, sp0_ref, sp1_ref, emb_ref, n1_ref,
             gcn_ref, hyp_ref, nx_ref, out_ref) = refs
        else:
            (p_ref, tmp_ref, sp0_ref, sp1_ref,
             gcn_ref, hyp_ref, nx_ref) = refs
        h = jnp.dot(p_ref[...], tmp_ref[0], preferred_element_type=jnp.float32)
        h = _l2n(h)
        g = _l2n(jnp.concatenate([sp0_ref[...], sp1_ref[...]], axis=1))
        gcn_ref[...] = g
        hyp_ref[...] = h
        nxt = g + h
        nx_ref[...] = nxt
        if final:
            out_ref[...] = emb_ref[...] + n1_ref[...] + nxt

    spec_rb_h = pl.BlockSpec((RB, H), lambda i: (i, 0))
    spec_rb_d = pl.BlockSpec((RB, D), lambda i: (i, 0))
    in_specs = [spec_rb_h,
                pl.BlockSpec((1, H, D), lambda i: (_type_of_block(i), 0, 0)),
                spec_rb_h, spec_rb_h]
    out_specs = [spec_rb_d, spec_rb_d, spec_rb_d]
    out_shape = [jax.ShapeDtypeStruct((N, D), jnp.float32),
                 jax.ShapeDtypeStruct((N, D), jnp.float32),
                 jax.ShapeDtypeStruct((N, D), jnp.float32)]
    args = [p, tmp, sp0, sp1]
    if final:
        in_specs += [spec_rb_d, spec_rb_d]
        out_specs = out_specs + [spec_rb_d]
        out_shape = out_shape + [jax.ShapeDtypeStruct((N, D), jnp.float32)]
        args += list(extras)

    return pl.pallas_call(
        body,
        grid=(NBLK,),
        in_specs=in_specs,
        out_specs=out_specs,
        out_shape=out_shape,
    )(*args)


def kernel(adj_indices, adj_values, keepRate, dEmbeds, mEmbeds, disEmbeds,
           dHyper, mHyper, disHyper):
    del keepRate
    src = adj_indices[1].astype(jnp.int32)
    dst = adj_indices[0].astype(jnp.int32)
    vals = adj_values.astype(jnp.float32)
    zeros = jnp.zeros((ART_LAST, H), jnp.float32)

    embeds = jnp.concatenate([dEmbeds, mEmbeds, disEmbeds], axis=0)
    hypers = jnp.stack([dHyper, mHyper, disHyper])

    p = _tc_init(embeds, hypers)

    csrc, cdst, cval, cnts = _get_sc("part")(src, dst, vals)
    spmm_fn = _get_sc("spmm")

    s0, s1 = spmm_fn(csrc, cdst, cval, cnts, zeros, _pack_rows(embeds))
    tmp = _tc_phase_a(p, embeds)
    gcn1, hyper1, n1 = _tc_phase_b(p, tmp, s0, s1)

    s0, s1 = spmm_fn(csrc, cdst, cval, cnts, zeros, _pack_rows(n1))
    tmp = _tc_phase_a(p, n1)
    gcn2, hyper2, _, out = _tc_phase_b(p, tmp, s0, s1, extras=(embeds, n1))

    return (out, embeds, gcn1, gcn2, embeds, hyper1, hyper2)
